# baseline (speedup 1.0000x reference)
import functools

import jax
import jax.numpy as jnp
from jax import lax
from jax.experimental import pallas as pl
from jax.experimental.pallas import tpu as pltpu
from jax.experimental.pallas import tpu_sc as plsc

B = 16384
D = 64
NC = 2
NS = 16
L = 16
NW = NC * NS
BPW = B // NW
GROUPS = BPW // L


def _bpr_body(users_hbm, pos_hbm, neg_hbm, uemb_hbm, iemb_hbm, ibias_hbm,
              out_hbm,
              uidx_v, pidx_v, nidx_v, urows_v, prows_v, nrows_v,
              pb_v, nb_v, out_v,
              sem_u, sem_p, sem_n, sem_pb, sem_nb):
    wid = lax.axis_index("s") * NC + lax.axis_index("c")
    base = wid * BPW

    pltpu.sync_copy(users_hbm.at[pl.ds(base, BPW)], uidx_v)
    pltpu.sync_copy(pos_hbm.at[pl.ds(base, BPW)], pidx_v)
    pltpu.sync_copy(neg_hbm.at[pl.ds(base, BPW)], nidx_v)

    cu = pltpu.async_copy(uemb_hbm.at[uidx_v], urows_v, sem_u)
    cp = pltpu.async_copy(iemb_hbm.at[pidx_v], prows_v, sem_p)
    cn = pltpu.async_copy(iemb_hbm.at[nidx_v], nrows_v, sem_n)
    cpb = pltpu.async_copy(ibias_hbm.at[pidx_v], pb_v, sem_pb)
    cnb = pltpu.async_copy(ibias_hbm.at[nidx_v], nb_v, sem_nb)
    cu.wait()
    cp.wait()
    cn.wait()
    cpb.wait()
    cnb.wait()

    lane = lax.iota(jnp.int32, L)

    def group(g, carry):
        rows = g * L + lane
        acc = pb_v[pl.ds(g * L, L)] - nb_v[pl.ds(g * L, L)]
        for d in range(D):
            col = jnp.full((L,), d, jnp.int32)
            u = plsc.load_gather(urows_v, [rows, col])
            p = plsc.load_gather(prows_v, [rows, col])
            n = plsc.load_gather(nrows_v, [rows, col])
            acc = acc + u * (p - n)
        out_v[pl.ds(g * L, L)] = acc
        return carry

    lax.fori_loop(0, GROUPS, group, 0, unroll=False)

    pltpu.sync_copy(out_v, out_hbm.at[pl.ds(base, BPW)])


@functools.partial(jax.jit, static_argnames=())
def _bpr_call(users, pos_items, neg_items, user_embeddings, item_embeddings,
              item_biases_flat):
    mesh = plsc.VectorSubcoreMesh(core_axis_name="c", subcore_axis_name="s")
    return pl.kernel(
        _bpr_body,
        out_type=jax.ShapeDtypeStruct((B,), jnp.float32),
        mesh=mesh,
        compiler_params=pltpu.CompilerParams(
            needs_layout_passes=False, use_tc_tiling_on_sc=False),
        scratch_types=[
            pltpu.VMEM((BPW,), jnp.int32),
            pltpu.VMEM((BPW,), jnp.int32),
            pltpu.VMEM((BPW,), jnp.int32),
            pltpu.VMEM((BPW, D), jnp.float32),
            pltpu.VMEM((BPW, D), jnp.float32),
            pltpu.VMEM((BPW, D), jnp.float32),
            pltpu.VMEM((BPW,), jnp.float32),
            pltpu.VMEM((BPW,), jnp.float32),
            pltpu.VMEM((BPW,), jnp.float32),
            pltpu.SemaphoreType.DMA,
            pltpu.SemaphoreType.DMA,
            pltpu.SemaphoreType.DMA,
            pltpu.SemaphoreType.DMA,
            pltpu.SemaphoreType.DMA,
        ],
    )(users, pos_items, neg_items, user_embeddings, item_embeddings,
      item_biases_flat)


def kernel(users, pos_items, neg_items, user_embeddings, item_embeddings,
           user_biases, item_biases):
    del user_biases
    return _bpr_call(users, pos_items, neg_items, user_embeddings,
                     item_embeddings, item_biases.reshape(-1))


# speedup vs baseline: 1.4465x; 1.4465x over previous
import jax
import jax.numpy as jnp
from jax import lax
from jax.experimental import pallas as pl
from jax.experimental.pallas import tpu as pltpu
from jax.experimental.pallas import tpu_sc as plsc

B = 16384
D = 64
NC = 2
NS = 16
L = 16
NW = NC * NS
BPW = B // NW
CHUNK = 128
NPASS = BPW // CHUNK
CGROUPS = CHUNK // L


def _bpr_body(users_hbm, pos_hbm, neg_hbm, uemb_hbm, iemb_hbm, ibias_hbm,
              out_hbm,
              uidx_v, pidx_v, nidx_v, urows_v, prows_v, nrows_v,
              pb_v, nb_v, out_v,
              sem_rows, sem_bias):
    wid = lax.axis_index("s") * NC + lax.axis_index("c")
    base = wid * BPW

    pltpu.sync_copy(users_hbm.at[pl.ds(base, BPW)], uidx_v)
    pltpu.sync_copy(pos_hbm.at[pl.ds(base, BPW)], pidx_v)
    pltpu.sync_copy(neg_hbm.at[pl.ds(base, BPW)], nidx_v)

    lane = lax.iota(jnp.int32, L)

    def fire(p):
        slot = p % 2
        off = p * CHUNK
        ur = urows_v.at[slot]
        pr = prows_v.at[slot]
        nr = nrows_v.at[slot]

        def fire_group(g, carry):
            gb = off + g * L
            uvec = uidx_v[pl.ds(gb, L)]
            pvec = pidx_v[pl.ds(gb, L)]
            nvec = nidx_v[pl.ds(gb, L)]
            for jj in range(L):
                j = g * L + jj
                pltpu.async_copy(uemb_hbm.at[uvec[jj]], ur.at[j],
                                 sem_rows.at[slot])
                pltpu.async_copy(iemb_hbm.at[pvec[jj]], pr.at[j],
                                 sem_rows.at[slot])
                pltpu.async_copy(iemb_hbm.at[nvec[jj]], nr.at[j],
                                 sem_rows.at[slot])
            return carry

        lax.fori_loop(0, CGROUPS, fire_group, 0, unroll=False)
        cpb = pltpu.async_copy(
            ibias_hbm.at[pidx_v.at[pl.ds(off, CHUNK)]], pb_v.at[slot],
            sem_bias.at[slot])
        cnb = pltpu.async_copy(
            ibias_hbm.at[nidx_v.at[pl.ds(off, CHUNK)]], nb_v.at[slot],
            sem_bias.at[slot])
        return cpb, cnb

    def drain(p, cpb, cnb):
        slot = p % 2
        pltpu.make_async_copy(uemb_hbm.at[pl.ds(0, CHUNK)],
                              urows_v.at[slot], sem_rows.at[slot]).wait()
        pltpu.make_async_copy(iemb_hbm.at[pl.ds(0, CHUNK)],
                              prows_v.at[slot], sem_rows.at[slot]).wait()
        pltpu.make_async_copy(iemb_hbm.at[pl.ds(0, CHUNK)],
                              nrows_v.at[slot], sem_rows.at[slot]).wait()
        cpb.wait()
        cnb.wait()

    def compute(p):
        slot = p % 2
        off = p * CHUNK
        ur = urows_v.at[slot]
        pr = prows_v.at[slot]
        nr = nrows_v.at[slot]

        def group(g, carry):
            rows = g * L + lane
            acc = (pb_v[slot, pl.ds(g * L, L)]
                   - nb_v[slot, pl.ds(g * L, L)])
            for d in range(D):
                col = jnp.full((L,), d, jnp.int32)
                u = plsc.load_gather(ur, [rows, col])
                pp = plsc.load_gather(pr, [rows, col])
                nn = plsc.load_gather(nr, [rows, col])
                acc = acc + u * (pp - nn)
            out_v[pl.ds(off + g * L, L)] = acc
            return carry

        lax.fori_loop(0, CGROUPS, group, 0, unroll=False)

    pending = fire(0)
    for p in range(NPASS):
        nxt = fire(p + 1) if p + 1 < NPASS else None
        drain(p, *pending)
        compute(p)
        pending = nxt

    pltpu.sync_copy(out_v, out_hbm.at[pl.ds(base, BPW)])


@jax.jit
def _bpr_call(users, pos_items, neg_items, user_embeddings, item_embeddings,
              item_biases_flat):
    mesh = plsc.VectorSubcoreMesh(core_axis_name="c", subcore_axis_name="s")
    return pl.kernel(
        _bpr_body,
        out_type=jax.ShapeDtypeStruct((B,), jnp.float32),
        mesh=mesh,
        compiler_params=pltpu.CompilerParams(
            needs_layout_passes=False, use_tc_tiling_on_sc=True),
        scratch_types=[
            pltpu.VMEM((BPW,), jnp.int32),
            pltpu.VMEM((BPW,), jnp.int32),
            pltpu.VMEM((BPW,), jnp.int32),
            pltpu.VMEM((2, CHUNK, D), jnp.float32),
            pltpu.VMEM((2, CHUNK, D), jnp.float32),
            pltpu.VMEM((2, CHUNK, D), jnp.float32),
            pltpu.VMEM((2, CHUNK), jnp.float32),
            pltpu.VMEM((2, CHUNK), jnp.float32),
            pltpu.VMEM((BPW,), jnp.float32),
            pltpu.SemaphoreType.DMA((2,)),
            pltpu.SemaphoreType.DMA((2,)),
        ],
    )(users, pos_items, neg_items, user_embeddings, item_embeddings,
      item_biases_flat)


def kernel(users, pos_items, neg_items, user_embeddings, item_embeddings,
           user_biases, item_biases):
    del user_biases
    return _bpr_call(users, pos_items, neg_items, user_embeddings,
                     item_embeddings, item_biases.reshape(-1))
